# Initial kernel scaffold; baseline (speedup 1.0000x reference)
import jax
import jax.numpy as jnp
from jax.experimental import pallas as pl


def kernel(x, edge_index, edge_weight, W1, b1, W2, b2, W_head, b_head):
    raise NotImplementedError("write your pallas kernel here")


# speedup vs baseline: 8.9053x; 8.9053x over previous
import functools

import jax
import jax.numpy as jnp
from jax import lax
from jax.experimental import pallas as pl
from jax.experimental.pallas import tpu as pltpu
from jax.experimental.pallas import tpu_sc as plsc

N = 10000
E = 160000
D = 256
HD = 128
CLS = 64
NC, NS = 2, 16

A_CH, A_K = 125, 80
D_CH, D_K = 125, 40
STRIPE = 640
LAST = N - (NS - 1) * STRIPE

ROWS = 400
GRID = N // ROWS


def _striped(copy_fn, s):
    off = pl.multiple_of(s * STRIPE, 8)

    @pl.when(s < NS - 1)
    def _():
        copy_fn(off, STRIPE)

    @pl.when(s == NS - 1)
    def _():
        copy_fn((NS - 1) * STRIPE, LAST)


def _staged_1d(src_at, dst_at, stage, s):
    def cp(o, n):
        pltpu.sync_copy(src_at(o, n), stage.at[pl.ds(0, n)])
        pltpu.sync_copy(stage.at[pl.ds(0, n)], dst_at(o, n))

    _striped(cp, s)


def _staged_2d(src_at, dst_at, stage, s):
    def cp(o, n):
        for k in range(n // A_K):
            po = pl.multiple_of(o + k * A_K, 8)
            pltpu.sync_copy(src_at(po, A_K), stage)
            pltpu.sync_copy(stage, dst_at(po, A_K))

    _striped(cp, s)


def _deg_body(dst_hbm, ew_hbm, init_hbm, outa_hbm, outb_hbm,
              acc, dst_v, ew_v, stage_v):
    c = lax.axis_index("c")
    s = lax.axis_index("s")
    w = c * NS + s

    _staged_1d(lambda o, n: init_hbm.at[pl.ds(o, n)],
               lambda o, n: acc.at[pl.ds(o, n)], stage_v, s)
    pltpu.sync_copy(dst_hbm.at[w], dst_v)
    pltpu.sync_copy(ew_hbm.at[w], ew_v)
    plsc.subcore_barrier()

    def chunk(j, carry):
        pltpu.sync_copy(ew_v.at[j], acc.at[dst_v.at[j]], add=True)
        return carry

    lax.fori_loop(0, D_CH, chunk, 0)
    plsc.subcore_barrier()

    @pl.when(c == 0)
    def _():
        _staged_1d(lambda o, n: acc.at[pl.ds(o, n)],
                   lambda o, n: outa_hbm.at[pl.ds(o, n)], stage_v, s)

    @pl.when(c == 1)
    def _():
        _staged_1d(lambda o, n: acc.at[pl.ds(o, n)],
                   lambda o, n: outb_hbm.at[pl.ds(o, n)], stage_v, s)


_deg_call = pl.kernel(
    _deg_body,
    out_type=(jax.ShapeDtypeStruct((N,), jnp.float32),
              jax.ShapeDtypeStruct((N,), jnp.float32)),
    mesh=plsc.VectorSubcoreMesh(core_axis_name="c", subcore_axis_name="s"),
    scratch_types=[
        pltpu.VMEM_SHARED((N,), jnp.float32),
        pltpu.VMEM((D_CH, D_K), jnp.int32),
        pltpu.VMEM((D_CH, D_K), jnp.float32),
        pltpu.VMEM((STRIPE,), jnp.float32),
    ],
)


_GD = lax.GatherDimensionNumbers(offset_dims=(), collapsed_slice_dims=(0,),
                                 start_index_map=(0,))


def _splat16(vec16, e):
    idx = jnp.full((16, 1), e, jnp.int32)
    return lax.gather(vec16, idx, _GD, (1,),
                      mode=lax.GatherScatterMode.PROMISE_IN_BOUNDS)


def _agg_half(hp_hbm, out_hbm, s, acc, src_v, dst_v, ew_v, gbuf):
    _staged_2d(lambda o, n: hp_hbm.at[pl.ds(o, n)],
               lambda o, n: acc.at[pl.ds(o, n)], gbuf, s)
    plsc.subcore_barrier()

    def chunk(j, carry):
        eoff = pl.multiple_of(j * A_K, 8)
        pltpu.sync_copy(hp_hbm.at[src_v.at[pl.ds(eoff, A_K)]], gbuf)
        for g in range(A_K // 16):
            ewg = ew_v[pl.ds(pl.multiple_of(j * A_K + g * 16, 8), 16)]
            for e in range(16):
                splat = _splat16(ewg, e)
                r = g * 16 + e
                for f in range(HD // 16):
                    gbuf[r, pl.ds(f * 16, 16)] = (
                        gbuf[r, pl.ds(f * 16, 16)] * splat)
        pltpu.sync_copy(gbuf, acc.at[dst_v.at[j]], add=True)
        return carry

    lax.fori_loop(0, A_CH, chunk, 0)
    plsc.subcore_barrier()
    _staged_2d(lambda o, n: acc.at[pl.ds(o, n)],
               lambda o, n: out_hbm.at[pl.ds(o, n)], gbuf, s)


def _agg_body(src_hbm, dst_hbm, ew_hbm, hpa_hbm, hpb_hbm,
              outa_hbm, outb_hbm, acc, src_v, dst_v, ew_v, gbuf):
    c = lax.axis_index("c")
    s = lax.axis_index("s")
    pltpu.sync_copy(src_hbm.at[s], src_v)
    pltpu.sync_copy(dst_hbm.at[s], dst_v)
    pltpu.sync_copy(ew_hbm.at[s], ew_v)

    @pl.when(c == 0)
    def _():
        _agg_half(hpa_hbm, outa_hbm, s, acc, src_v, dst_v, ew_v, gbuf)

    @pl.when(c == 1)
    def _():
        _agg_half(hpb_hbm, outb_hbm, s, acc, src_v, dst_v, ew_v, gbuf)


_agg_call = pl.kernel(
    _agg_body,
    out_type=(jax.ShapeDtypeStruct((N, HD), jnp.float32),
              jax.ShapeDtypeStruct((N, HD), jnp.float32)),
    mesh=plsc.VectorSubcoreMesh(core_axis_name="c", subcore_axis_name="s"),
    scratch_types=[
        pltpu.VMEM_SHARED((N, HD), jnp.float32),
        pltpu.VMEM((E // NS,), jnp.int32),
        pltpu.VMEM((A_CH, A_K), jnp.int32),
        pltpu.VMEM((E // NS,), jnp.float32),
        pltpu.VMEM((A_K, HD), jnp.float32),
    ],
)


def _mm1_body(x_ref, w_ref, dega_ref, degb_ref, outa_ref, outb_ref):
    dis = lax.rsqrt(dega_ref[:, 0] + degb_ref[:, 0])
    h = jnp.dot(x_ref[...], w_ref[...], preferred_element_type=jnp.float32,
                precision=lax.Precision.HIGHEST)
    hp = dis[:, None] * h
    outa_ref[...] = hp[:, :HD]
    outb_ref[...] = hp[:, HD:]


_mm1_call = pl.pallas_call(
    _mm1_body,
    grid=(GRID,),
    in_specs=[
        pl.BlockSpec((ROWS, D), lambda i: (i, 0)),
        pl.BlockSpec((D, D), lambda i: (0, 0)),
        pl.BlockSpec((ROWS, 1), lambda i: (i, 0)),
        pl.BlockSpec((ROWS, 1), lambda i: (i, 0)),
    ],
    out_specs=(pl.BlockSpec((ROWS, HD), lambda i: (i, 0)),
               pl.BlockSpec((ROWS, HD), lambda i: (i, 0))),
    out_shape=(jax.ShapeDtypeStruct((N, HD), jnp.float32),
               jax.ShapeDtypeStruct((N, HD), jnp.float32)),
)


def _mm2_body(sa_ref, sb_ref, dega_ref, degb_ref, b_ref, w_ref,
              outa_ref, outb_ref):
    dis = lax.rsqrt(dega_ref[:, 0] + degb_ref[:, 0])
    za = jnp.maximum(dis[:, None] * sa_ref[...] + b_ref[0, :HD], 0.0)
    zb = jnp.maximum(dis[:, None] * sb_ref[...] + b_ref[0, HD:], 0.0)
    z = jnp.concatenate([za, zb], axis=1)
    h = jnp.dot(z, w_ref[...], preferred_element_type=jnp.float32,
                precision=lax.Precision.HIGHEST)
    hp = dis[:, None] * h
    outa_ref[...] = hp[:, :HD]
    outb_ref[...] = hp[:, HD:]


_mm2_call = pl.pallas_call(
    _mm2_body,
    grid=(GRID,),
    in_specs=[
        pl.BlockSpec((ROWS, HD), lambda i: (i, 0)),
        pl.BlockSpec((ROWS, HD), lambda i: (i, 0)),
        pl.BlockSpec((ROWS, 1), lambda i: (i, 0)),
        pl.BlockSpec((ROWS, 1), lambda i: (i, 0)),
        pl.BlockSpec((1, D), lambda i: (0, 0)),
        pl.BlockSpec((D, D), lambda i: (0, 0)),
    ],
    out_specs=(pl.BlockSpec((ROWS, HD), lambda i: (i, 0)),
               pl.BlockSpec((ROWS, HD), lambda i: (i, 0))),
    out_shape=(jax.ShapeDtypeStruct((N, HD), jnp.float32),
               jax.ShapeDtypeStruct((N, HD), jnp.float32)),
)


def _head_body(sa_ref, sb_ref, dega_ref, degb_ref, b_ref, wh_ref, bh_ref,
               lg_ref, sm_ref):
    dis = lax.rsqrt(dega_ref[:, 0] + degb_ref[:, 0])
    za = jnp.maximum(dis[:, None] * sa_ref[...] + b_ref[0, :HD], 0.0)
    zb = jnp.maximum(dis[:, None] * sb_ref[...] + b_ref[0, HD:], 0.0)
    z = jnp.concatenate([za, zb], axis=1)
    lg = jnp.dot(z, wh_ref[...], preferred_element_type=jnp.float32,
                 precision=lax.Precision.HIGHEST) + bh_ref[0]
    lg_ref[...] = lg
    m = jnp.max(lg, axis=1, keepdims=True)
    ex = jnp.exp(lg - m)
    sm_ref[...] = ex / jnp.sum(ex, axis=1, keepdims=True)


_head_call = pl.pallas_call(
    _head_body,
    grid=(GRID,),
    in_specs=[
        pl.BlockSpec((ROWS, HD), lambda i: (i, 0)),
        pl.BlockSpec((ROWS, HD), lambda i: (i, 0)),
        pl.BlockSpec((ROWS, 1), lambda i: (i, 0)),
        pl.BlockSpec((ROWS, 1), lambda i: (i, 0)),
        pl.BlockSpec((1, D), lambda i: (0, 0)),
        pl.BlockSpec((D, CLS), lambda i: (0, 0)),
        pl.BlockSpec((1, CLS), lambda i: (0, 0)),
    ],
    out_specs=(pl.BlockSpec((ROWS, CLS), lambda i: (i, 0)),
               pl.BlockSpec((ROWS, CLS), lambda i: (i, 0))),
    out_shape=(jax.ShapeDtypeStruct((N, CLS), jnp.float32),
               jax.ShapeDtypeStruct((N, CLS), jnp.float32)),
)


def kernel(x, edge_index, edge_weight, W1, b1, W2, b2, W_head, b_head):
    src = edge_index[0]
    dst = edge_index[1]
    dstD = dst.reshape(NC * NS, D_CH, D_K)
    ewD = edge_weight.reshape(NC * NS, D_CH, D_K)
    srcA = src.reshape(NS, E // NS)
    dstA = dst.reshape(NS, A_CH, A_K)
    ewA = edge_weight.reshape(NS, E // NS)
    init_h = jnp.full((N,), 0.5, jnp.float32)

    dega, degb = _deg_call(dstD, ewD, init_h)
    dega = dega.reshape(N, 1)
    degb = degb.reshape(N, 1)

    b1r = b1.reshape(1, D)
    b2r = b2.reshape(1, D)
    bhr = b_head.reshape(1, CLS)

    hp1a, hp1b = _mm1_call(x, W1, dega, degb)
    s1a, s1b = _agg_call(srcA, dstA, ewA, hp1a, hp1b)
    hp2a, hp2b = _mm2_call(s1a, s1b, dega, degb, b1r, W2)
    s2a, s2b = _agg_call(srcA, dstA, ewA, hp2a, hp2b)
    logits, soft = _head_call(s2a, s2b, dega, degb, b2r, W_head, bhr)
    return (logits, soft)


# speedup vs baseline: 13.7758x; 1.5469x over previous
import functools

import numpy as _np

import jax
import jax.numpy as jnp
from jax import lax
from jax.experimental import pallas as pl
from jax.experimental.pallas import tpu as pltpu
from jax.experimental.pallas import tpu_sc as plsc

N = 10000
E = 160000
D = 256
HD = 128
CLS = 64
NC, NS = 2, 16

A_CH, A_K = 125, 80
D_CH, D_K = 125, 40
STRIPE = 640
LAST = N - (NS - 1) * STRIPE

ROWS = 400
GRID = N // ROWS


def _striped(copy_fn, s):
    off = pl.multiple_of(s * STRIPE, 8)

    @pl.when(s < NS - 1)
    def _():
        copy_fn(off, STRIPE)

    @pl.when(s == NS - 1)
    def _():
        copy_fn((NS - 1) * STRIPE, LAST)


def _staged_1d(src_at, dst_at, stage, s):
    def cp(o, n):
        pltpu.sync_copy(src_at(o, n), stage.at[pl.ds(0, n)])
        pltpu.sync_copy(stage.at[pl.ds(0, n)], dst_at(o, n))

    _striped(cp, s)


def _staged_2d(src_at, dst_at, stage, s):
    def cp(o, n):
        for k in range(n // A_K):
            po = pl.multiple_of(o + k * A_K, 8)
            pltpu.sync_copy(src_at(po, A_K), stage)
            pltpu.sync_copy(stage, dst_at(po, A_K))

    _striped(cp, s)


def _deg_body(dst_hbm, ew_hbm, init_hbm, outa_hbm, outb_hbm,
              acc, dst_v, ew_v, stage_v, sem):
    c = lax.axis_index("c")
    s = lax.axis_index("s")
    w = c * NS + s

    _staged_1d(lambda o, n: init_hbm.at[pl.ds(o, n)],
               lambda o, n: acc.at[pl.ds(o, n)], stage_v, s)
    pltpu.sync_copy(dst_hbm.at[w], dst_v)
    pltpu.sync_copy(ew_hbm.at[w], ew_v)
    plsc.subcore_barrier()

    def group(jg, carry):
        for k in range(5):
            j = jg * 5 + k
            pltpu.async_copy(ew_v.at[j], acc.at[dst_v.at[j]], sem, add=True)
        for k in range(5):
            j = jg * 5 + k
            pltpu.make_async_copy(ew_v.at[j], acc.at[dst_v.at[j]],
                                  sem).wait()
        return carry

    lax.fori_loop(0, D_CH // 5, group, 0)
    plsc.subcore_barrier()

    @pl.when(c == 0)
    def _():
        _staged_1d(lambda o, n: acc.at[pl.ds(o, n)],
                   lambda o, n: outa_hbm.at[pl.ds(o, n)], stage_v, s)

    @pl.when(c == 1)
    def _():
        _staged_1d(lambda o, n: acc.at[pl.ds(o, n)],
                   lambda o, n: outb_hbm.at[pl.ds(o, n)], stage_v, s)


_deg_call = pl.kernel(
    _deg_body,
    out_type=(jax.ShapeDtypeStruct((N,), jnp.float32),
              jax.ShapeDtypeStruct((N,), jnp.float32)),
    mesh=plsc.VectorSubcoreMesh(core_axis_name="c", subcore_axis_name="s"),
    scratch_types=[
        pltpu.VMEM_SHARED((N,), jnp.float32),
        pltpu.VMEM((D_CH, D_K), jnp.int32),
        pltpu.VMEM((D_CH, D_K), jnp.float32),
        pltpu.VMEM((STRIPE,), jnp.float32),
        pltpu.SemaphoreType.DMA,
    ],
)


_GD = lax.GatherDimensionNumbers(offset_dims=(), collapsed_slice_dims=(0,),
                                 start_index_map=(0,))


def _splat16(vec16, e):
    idx = jnp.full((16, 1), e, jnp.int32)
    return lax.gather(vec16, idx, _GD, (1,),
                      mode=lax.GatherScatterMode.PROMISE_IN_BOUNDS)


NBUF = 3


def _agg_half(hp_hbm, dst_hbm, ew_hbm, out_hbm, s, acc, src_v,
              gb, ering, dring, gsems, isems):
    _staged_2d(lambda o, n: hp_hbm.at[pl.ds(o, n)],
               lambda o, n: acc.at[pl.ds(o, n)], gb[0], s)
    plsc.subcore_barrier()

    def gstart(j, slot):
        eoff = pl.multiple_of(j * A_K, 8)
        pltpu.async_copy(hp_hbm.at[src_v.at[pl.ds(eoff, A_K)]],
                         gb[slot], gsems[slot])

    def gwait(slot):
        pltpu.make_async_copy(hp_hbm.at[src_v.at[pl.ds(0, A_K)]],
                              gb[slot], gsems[slot]).wait()

    def istart(j, slot):
        pltpu.async_copy(dst_hbm.at[s, j], dring.at[slot], isems[slot])
        pltpu.async_copy(ew_hbm.at[s, j], ering.at[slot], isems[slot])

    def iwait(slot):
        pltpu.make_async_copy(dst_hbm.at[s, 0], dring.at[slot],
                              isems[slot]).wait()
        pltpu.make_async_copy(ew_hbm.at[s, 0], ering.at[slot],
                              isems[slot]).wait()

    def process(j, slot):
        gwait(slot)
        iwait(slot)
        buf = gb[slot]

        @pl.loop(0, A_K // 16)
        def _scale(g):
            ewg = ering[slot, pl.ds(pl.multiple_of(g * 16, 16), 16)]
            for e in range(16):
                splat = _splat16(ewg, e)
                r = g * 16 + e
                for f in range(HD // 16):
                    buf[r, pl.ds(f * 16, 16)] = (
                        buf[r, pl.ds(f * 16, 16)] * splat)

        pltpu.sync_copy(buf, acc.at[dring.at[slot]], add=True)

    for p in range(NBUF - 1):
        istart(p, p)
        gstart(p, p)

    def body(j, carry):
        for b in range(NBUF):
            @pl.when(j % NBUF == b)
            def _():
                process(j, b)

                @pl.when(j < A_CH - (NBUF - 1))
                def _(nslot=(b + NBUF - 1) % NBUF):
                    istart(j + NBUF - 1, nslot)
                    gstart(j + NBUF - 1, nslot)

        return carry

    lax.fori_loop(0, A_CH, body, 0)
    plsc.subcore_barrier()
    _staged_2d(lambda o, n: acc.at[pl.ds(o, n)],
               lambda o, n: out_hbm.at[pl.ds(o, n)], gb[0], s)


def _agg_body(src_hbm, dst_hbm, ew_hbm, hpa_hbm, hpb_hbm,
              outa_hbm, outb_hbm, acc, src_v, gb0, gb1, gb2,
              ering, dring, gs0, gs1, gs2, is0, is1, is2):
    c = lax.axis_index("c")
    s = lax.axis_index("s")
    pltpu.sync_copy(src_hbm.at[s], src_v)
    gb = (gb0, gb1, gb2)
    gsems = (gs0, gs1, gs2)
    isems = (is0, is1, is2)

    @pl.when(c == 0)
    def _():
        _agg_half(hpa_hbm, dst_hbm, ew_hbm, outa_hbm, s, acc, src_v,
                  gb, ering, dring, gsems, isems)

    @pl.when(c == 1)
    def _():
        _agg_half(hpb_hbm, dst_hbm, ew_hbm, outb_hbm, s, acc, src_v,
                  gb, ering, dring, gsems, isems)


_agg_call = pl.kernel(
    _agg_body,
    out_type=(jax.ShapeDtypeStruct((N, HD), jnp.float32),
              jax.ShapeDtypeStruct((N, HD), jnp.float32)),
    mesh=plsc.VectorSubcoreMesh(core_axis_name="c", subcore_axis_name="s"),
    scratch_types=[
        pltpu.VMEM_SHARED((N, HD), jnp.float32),
        pltpu.VMEM((E // NS,), jnp.int32),
        pltpu.VMEM((A_K, HD), jnp.float32),
        pltpu.VMEM((A_K, HD), jnp.float32),
        pltpu.VMEM((A_K, HD), jnp.float32),
        pltpu.VMEM((NBUF, A_K), jnp.float32),
        pltpu.VMEM((NBUF, A_K), jnp.int32),
        pltpu.SemaphoreType.DMA,
        pltpu.SemaphoreType.DMA,
        pltpu.SemaphoreType.DMA,
        pltpu.SemaphoreType.DMA,
        pltpu.SemaphoreType.DMA,
        pltpu.SemaphoreType.DMA,
    ],
)


def _mm1_body(x_ref, w_ref, dega_ref, degb_ref, outa_ref, outb_ref):
    dis = lax.rsqrt(dega_ref[:, 0] + degb_ref[:, 0])
    h = jnp.dot(x_ref[...], w_ref[...], preferred_element_type=jnp.float32,
                precision=lax.Precision.HIGHEST)
    hp = dis[:, None] * h
    outa_ref[...] = hp[:, :HD]
    outb_ref[...] = hp[:, HD:]


_mm1_call = pl.pallas_call(
    _mm1_body,
    grid=(GRID,),
    in_specs=[
        pl.BlockSpec((ROWS, D), lambda i: (i, 0)),
        pl.BlockSpec((D, D), lambda i: (0, 0)),
        pl.BlockSpec((ROWS, 1), lambda i: (i, 0)),
        pl.BlockSpec((ROWS, 1), lambda i: (i, 0)),
    ],
    out_specs=(pl.BlockSpec((ROWS, HD), lambda i: (i, 0)),
               pl.BlockSpec((ROWS, HD), lambda i: (i, 0))),
    out_shape=(jax.ShapeDtypeStruct((N, HD), jnp.float32),
               jax.ShapeDtypeStruct((N, HD), jnp.float32)),
)


def _mm2_body(sa_ref, sb_ref, dega_ref, degb_ref, b_ref, w_ref,
              outa_ref, outb_ref):
    dis = lax.rsqrt(dega_ref[:, 0] + degb_ref[:, 0])
    za = jnp.maximum(dis[:, None] * sa_ref[...] + b_ref[0, :HD], 0.0)
    zb = jnp.maximum(dis[:, None] * sb_ref[...] + b_ref[0, HD:], 0.0)
    z = jnp.concatenate([za, zb], axis=1)
    h = jnp.dot(z, w_ref[...], preferred_element_type=jnp.float32,
                precision=lax.Precision.HIGHEST)
    hp = dis[:, None] * h
    outa_ref[...] = hp[:, :HD]
    outb_ref[...] = hp[:, HD:]


_mm2_call = pl.pallas_call(
    _mm2_body,
    grid=(GRID,),
    in_specs=[
        pl.BlockSpec((ROWS, HD), lambda i: (i, 0)),
        pl.BlockSpec((ROWS, HD), lambda i: (i, 0)),
        pl.BlockSpec((ROWS, 1), lambda i: (i, 0)),
        pl.BlockSpec((ROWS, 1), lambda i: (i, 0)),
        pl.BlockSpec((1, D), lambda i: (0, 0)),
        pl.BlockSpec((D, D), lambda i: (0, 0)),
    ],
    out_specs=(pl.BlockSpec((ROWS, HD), lambda i: (i, 0)),
               pl.BlockSpec((ROWS, HD), lambda i: (i, 0))),
    out_shape=(jax.ShapeDtypeStruct((N, HD), jnp.float32),
               jax.ShapeDtypeStruct((N, HD), jnp.float32)),
)


def _head_body(sa_ref, sb_ref, dega_ref, degb_ref, b_ref, wh_ref, bh_ref,
               lg_ref, sm_ref):
    dis = lax.rsqrt(dega_ref[:, 0] + degb_ref[:, 0])
    za = jnp.maximum(dis[:, None] * sa_ref[...] + b_ref[0, :HD], 0.0)
    zb = jnp.maximum(dis[:, None] * sb_ref[...] + b_ref[0, HD:], 0.0)
    z = jnp.concatenate([za, zb], axis=1)
    lg = jnp.dot(z, wh_ref[...], preferred_element_type=jnp.float32,
                 precision=lax.Precision.HIGHEST) + bh_ref[0]
    lg_ref[...] = lg
    m = jnp.max(lg, axis=1, keepdims=True)
    ex = jnp.exp(lg - m)
    sm_ref[...] = ex / jnp.sum(ex, axis=1, keepdims=True)


_head_call = pl.pallas_call(
    _head_body,
    grid=(GRID,),
    in_specs=[
        pl.BlockSpec((ROWS, HD), lambda i: (i, 0)),
        pl.BlockSpec((ROWS, HD), lambda i: (i, 0)),
        pl.BlockSpec((ROWS, 1), lambda i: (i, 0)),
        pl.BlockSpec((ROWS, 1), lambda i: (i, 0)),
        pl.BlockSpec((1, D), lambda i: (0, 0)),
        pl.BlockSpec((D, CLS), lambda i: (0, 0)),
        pl.BlockSpec((1, CLS), lambda i: (0, 0)),
    ],
    out_specs=(pl.BlockSpec((ROWS, CLS), lambda i: (i, 0)),
               pl.BlockSpec((ROWS, CLS), lambda i: (i, 0))),
    out_shape=(jax.ShapeDtypeStruct((N, CLS), jnp.float32),
               jax.ShapeDtypeStruct((N, CLS), jnp.float32)),
)


def kernel(x, edge_index, edge_weight, W1, b1, W2, b2, W_head, b_head):
    src = edge_index[0]
    dst = edge_index[1]
    dstD = dst.reshape(NC * NS, D_CH, D_K)
    ewD = edge_weight.reshape(NC * NS, D_CH, D_K)
    srcA = src.reshape(NS, E // NS)
    dstA = dst.reshape(NS, A_CH, A_K)
    ewA = edge_weight.reshape(NS, A_CH, A_K)
    init_h = jnp.full((N,), 0.5, jnp.float32)

    dega, degb = _deg_call(dstD, ewD, init_h)
    dega = dega.reshape(N, 1)
    degb = degb.reshape(N, 1)

    b1r = b1.reshape(1, D)
    b2r = b2.reshape(1, D)
    bhr = b_head.reshape(1, CLS)

    hp1a, hp1b = _mm1_call(x, W1, dega, degb)
    s1a, s1b = _agg_call(srcA, dstA, ewA, hp1a, hp1b)
    hp2a, hp2b = _mm2_call(s1a, s1b, dega, degb, b1r, W2)
    s2a, s2b = _agg_call(srcA, dstA, ewA, hp2a, hp2b)
    logits, soft = _head_call(s2a, s2b, dega, degb, b2r, W_head, bhr)
    return (logits, soft)


# speedup vs baseline: 15.4616x; 1.1224x over previous
import jax
import jax.numpy as jnp
from jax import lax
from jax.experimental import pallas as pl
from jax.experimental.pallas import tpu as pltpu
from jax.experimental.pallas import tpu_sc as plsc

N = 10000
E = 160000
D = 256
HD = 128
CLS = 64
NC, NS = 2, 16

A_CH, A_K = 125, 80
D_CH, D_K = 125, 40
STRIPE = 640
LAST = N - (NS - 1) * STRIPE

ROWS = 400
GRID = N // ROWS


def _striped(copy_fn, s):
    off = pl.multiple_of(s * STRIPE, 8)

    @pl.when(s < NS - 1)
    def _():
        copy_fn(off, STRIPE)

    @pl.when(s == NS - 1)
    def _():
        copy_fn((NS - 1) * STRIPE, LAST)


def _staged_1d(src_at, dst_at, stage, s):
    def cp(o, n):
        pltpu.sync_copy(src_at(o, n), stage.at[pl.ds(0, n)])
        pltpu.sync_copy(stage.at[pl.ds(0, n)], dst_at(o, n))

    _striped(cp, s)


def _staged_2d(src_at, dst_at, stage, s):
    def cp(o, n):
        for k in range(n // A_K):
            po = pl.multiple_of(o + k * A_K, 8)
            pltpu.sync_copy(src_at(po, A_K), stage)
            pltpu.sync_copy(stage, dst_at(po, A_K))

    _striped(cp, s)


def _staged_2d_pipe(src_at, dst_at, stages, sems, s):
    def cp(o, n):
        pieces = n // A_K

        def off(k):
            return pl.multiple_of(o + k * A_K, 8)

        pltpu.async_copy(src_at(off(0), A_K), stages[0], sems[0])
        for k in range(pieces):
            b = k % 2
            pltpu.make_async_copy(src_at(off(k), A_K), stages[b],
                                  sems[b]).wait()
            if k + 1 < pieces:
                pltpu.async_copy(src_at(off(k + 1), A_K), stages[1 - b],
                                 sems[1 - b])
            pltpu.sync_copy(stages[b], dst_at(off(k), A_K))

    _striped(cp, s)


def _deg_body(dst_hbm, ew_hbm, init_hbm, outa_hbm, outb_hbm,
              acc, dst_v, ew_v, stage_v, sem):
    c = lax.axis_index("c")
    s = lax.axis_index("s")
    w = c * NS + s

    _staged_1d(lambda o, n: init_hbm.at[pl.ds(o, n)],
               lambda o, n: acc.at[pl.ds(o, n)], stage_v, s)
    pltpu.sync_copy(dst_hbm.at[w], dst_v)
    pltpu.sync_copy(ew_hbm.at[w], ew_v)
    plsc.subcore_barrier()

    def group(jg, carry):
        for k in range(5):
            j = jg * 5 + k
            pltpu.async_copy(ew_v.at[j], acc.at[dst_v.at[j]], sem, add=True)
        for k in range(5):
            j = jg * 5 + k
            pltpu.make_async_copy(ew_v.at[j], acc.at[dst_v.at[j]],
                                  sem).wait()
        return carry

    lax.fori_loop(0, D_CH // 5, group, 0)
    plsc.subcore_barrier()

    @pl.when(c == 0)
    def _():
        _staged_1d(lambda o, n: acc.at[pl.ds(o, n)],
                   lambda o, n: outa_hbm.at[pl.ds(o, n)], stage_v, s)

    @pl.when(c == 1)
    def _():
        _staged_1d(lambda o, n: acc.at[pl.ds(o, n)],
                   lambda o, n: outb_hbm.at[pl.ds(o, n)], stage_v, s)


_deg_call = pl.kernel(
    _deg_body,
    out_type=(jax.ShapeDtypeStruct((N,), jnp.float32),
              jax.ShapeDtypeStruct((N,), jnp.float32)),
    mesh=plsc.VectorSubcoreMesh(core_axis_name="c", subcore_axis_name="s"),
    scratch_types=[
        pltpu.VMEM_SHARED((N,), jnp.float32),
        pltpu.VMEM((D_CH, D_K), jnp.int32),
        pltpu.VMEM((D_CH, D_K), jnp.float32),
        pltpu.VMEM((STRIPE,), jnp.float32),
        pltpu.SemaphoreType.DMA,
    ],
)


_GD = lax.GatherDimensionNumbers(offset_dims=(), collapsed_slice_dims=(0,),
                                 start_index_map=(0,))


def _splat16(vec16, e):
    idx = jnp.full((16, 1), e, jnp.int32)
    return lax.gather(vec16, idx, _GD, (1,),
                      mode=lax.GatherScatterMode.PROMISE_IN_BOUNDS)


NBUF = 3


def _agg_half(hp_hbm, dst_hbm, ew_hbm, out_hbm, s, acc, src_v,
              gb, ering, dring, gsems, isems, ssems):
    _staged_2d_pipe(lambda o, n: hp_hbm.at[pl.ds(o, n)],
                    lambda o, n: acc.at[pl.ds(o, n)],
                    (gb[0], gb[1]), (gsems[0], gsems[1]), s)
    plsc.subcore_barrier()

    def gstart(j, slot):
        eoff = pl.multiple_of(j * A_K, 8)
        pltpu.async_copy(hp_hbm.at[src_v.at[pl.ds(eoff, A_K)]],
                         gb[slot], gsems[slot])

    def gwait(slot):
        pltpu.make_async_copy(hp_hbm.at[src_v.at[pl.ds(0, A_K)]],
                              gb[slot], gsems[slot]).wait()

    def istart(j, slot):
        pltpu.async_copy(dst_hbm.at[s, j], dring.at[slot], isems[slot])
        pltpu.async_copy(ew_hbm.at[s, j], ering.at[slot], isems[slot])

    def iwait(slot):
        pltpu.make_async_copy(dst_hbm.at[s, 0], dring.at[slot],
                              isems[slot]).wait()
        pltpu.make_async_copy(ew_hbm.at[s, 0], ering.at[slot],
                              isems[slot]).wait()

    def swait(slot):
        pltpu.make_async_copy(gb[slot], acc.at[dring.at[slot]],
                              ssems[slot]).wait()

    def process(j, slot):
        gwait(slot)
        iwait(slot)
        buf = gb[slot]

        @pl.loop(0, A_K // 16)
        def _scale(g):
            ewg = ering[slot, pl.ds(pl.multiple_of(g * 16, 16), 16)]
            for e in range(16):
                splat = _splat16(ewg, e)
                r = g * 16 + e
                for f in range(HD // 16):
                    buf[r, pl.ds(f * 16, 16)] = (
                        buf[r, pl.ds(f * 16, 16)] * splat)

        pltpu.async_copy(buf, acc.at[dring.at[slot]], ssems[slot],
                         add=True)

    for p in range(NBUF - 1):
        istart(p, p)
        gstart(p, p)

    def body(j, carry):
        for b in range(NBUF):
            @pl.when(j % NBUF == b)
            def _():
                process(j, b)

                @pl.when(j < A_CH - (NBUF - 1))
                def _(nslot=(b + NBUF - 1) % NBUF):
                    @pl.when(j >= 1)
                    def _():
                        swait(nslot)

                    istart(j + NBUF - 1, nslot)
                    gstart(j + NBUF - 1, nslot)

        return carry

    lax.fori_loop(0, A_CH, body, 0)
    for k in range(NBUF):
        swait(k)
    plsc.subcore_barrier()
    _staged_2d_pipe(lambda o, n: acc.at[pl.ds(o, n)],
                    lambda o, n: out_hbm.at[pl.ds(o, n)],
                    (gb[0], gb[1]), (gsems[0], gsems[1]), s)


def _agg_body(src_hbm, dst_hbm, ew_hbm, hpa_hbm, hpb_hbm,
              outa_hbm, outb_hbm, acc, src_v, gb0, gb1, gb2,
              ering, dring, gs0, gs1, gs2, is0, is1, is2, ss0, ss1, ss2):
    c = lax.axis_index("c")
    s = lax.axis_index("s")
    pltpu.sync_copy(src_hbm.at[s], src_v)
    gb = (gb0, gb1, gb2)
    gsems = (gs0, gs1, gs2)
    isems = (is0, is1, is2)
    ssems = (ss0, ss1, ss2)

    @pl.when(c == 0)
    def _():
        _agg_half(hpa_hbm, dst_hbm, ew_hbm, outa_hbm, s, acc, src_v,
                  gb, ering, dring, gsems, isems, ssems)

    @pl.when(c == 1)
    def _():
        _agg_half(hpb_hbm, dst_hbm, ew_hbm, outb_hbm, s, acc, src_v,
                  gb, ering, dring, gsems, isems, ssems)


_agg_call = pl.kernel(
    _agg_body,
    out_type=(jax.ShapeDtypeStruct((N, HD), jnp.float32),
              jax.ShapeDtypeStruct((N, HD), jnp.float32)),
    mesh=plsc.VectorSubcoreMesh(core_axis_name="c", subcore_axis_name="s"),
    scratch_types=[
        pltpu.VMEM_SHARED((N, HD), jnp.float32),
        pltpu.VMEM((E // NS,), jnp.int32),
        pltpu.VMEM((A_K, HD), jnp.float32),
        pltpu.VMEM((A_K, HD), jnp.float32),
        pltpu.VMEM((A_K, HD), jnp.float32),
        pltpu.VMEM((NBUF, A_K), jnp.float32),
        pltpu.VMEM((NBUF, A_K), jnp.int32),
        pltpu.SemaphoreType.DMA,
        pltpu.SemaphoreType.DMA,
        pltpu.SemaphoreType.DMA,
        pltpu.SemaphoreType.DMA,
        pltpu.SemaphoreType.DMA,
        pltpu.SemaphoreType.DMA,
        pltpu.SemaphoreType.DMA,
        pltpu.SemaphoreType.DMA,
        pltpu.SemaphoreType.DMA,
    ],
)


def _mm1_body(x_ref, w_ref, dega_ref, degb_ref, outa_ref, outb_ref):
    dis = lax.rsqrt(dega_ref[:, 0] + degb_ref[:, 0])
    h = jnp.dot(x_ref[...], w_ref[...], preferred_element_type=jnp.float32,
                precision=lax.Precision.HIGHEST)
    hp = dis[:, None] * h
    outa_ref[...] = hp[:, :HD]
    outb_ref[...] = hp[:, HD:]


_mm1_call = pl.pallas_call(
    _mm1_body,
    grid=(GRID,),
    in_specs=[
        pl.BlockSpec((ROWS, D), lambda i: (i, 0)),
        pl.BlockSpec((D, D), lambda i: (0, 0)),
        pl.BlockSpec((ROWS, 1), lambda i: (i, 0)),
        pl.BlockSpec((ROWS, 1), lambda i: (i, 0)),
    ],
    out_specs=(pl.BlockSpec((ROWS, HD), lambda i: (i, 0)),
               pl.BlockSpec((ROWS, HD), lambda i: (i, 0))),
    out_shape=(jax.ShapeDtypeStruct((N, HD), jnp.float32),
               jax.ShapeDtypeStruct((N, HD), jnp.float32)),
)


def _mm2_body(sa_ref, sb_ref, dega_ref, degb_ref, b_ref, w_ref,
              outa_ref, outb_ref):
    dis = lax.rsqrt(dega_ref[:, 0] + degb_ref[:, 0])
    za = jnp.maximum(dis[:, None] * sa_ref[...] + b_ref[0, :HD], 0.0)
    zb = jnp.maximum(dis[:, None] * sb_ref[...] + b_ref[0, HD:], 0.0)
    z = jnp.concatenate([za, zb], axis=1)
    h = jnp.dot(z, w_ref[...], preferred_element_type=jnp.float32,
                precision=lax.Precision.HIGHEST)
    hp = dis[:, None] * h
    outa_ref[...] = hp[:, :HD]
    outb_ref[...] = hp[:, HD:]


_mm2_call = pl.pallas_call(
    _mm2_body,
    grid=(GRID,),
    in_specs=[
        pl.BlockSpec((ROWS, HD), lambda i: (i, 0)),
        pl.BlockSpec((ROWS, HD), lambda i: (i, 0)),
        pl.BlockSpec((ROWS, 1), lambda i: (i, 0)),
        pl.BlockSpec((ROWS, 1), lambda i: (i, 0)),
        pl.BlockSpec((1, D), lambda i: (0, 0)),
        pl.BlockSpec((D, D), lambda i: (0, 0)),
    ],
    out_specs=(pl.BlockSpec((ROWS, HD), lambda i: (i, 0)),
               pl.BlockSpec((ROWS, HD), lambda i: (i, 0))),
    out_shape=(jax.ShapeDtypeStruct((N, HD), jnp.float32),
               jax.ShapeDtypeStruct((N, HD), jnp.float32)),
)


def _head_body(sa_ref, sb_ref, dega_ref, degb_ref, b_ref, wh_ref, bh_ref,
               lg_ref, sm_ref):
    dis = lax.rsqrt(dega_ref[:, 0] + degb_ref[:, 0])
    za = jnp.maximum(dis[:, None] * sa_ref[...] + b_ref[0, :HD], 0.0)
    zb = jnp.maximum(dis[:, None] * sb_ref[...] + b_ref[0, HD:], 0.0)
    z = jnp.concatenate([za, zb], axis=1)
    lg = jnp.dot(z, wh_ref[...], preferred_element_type=jnp.float32,
                 precision=lax.Precision.HIGHEST) + bh_ref[0]
    lg_ref[...] = lg
    m = jnp.max(lg, axis=1, keepdims=True)
    ex = jnp.exp(lg - m)
    sm_ref[...] = ex / jnp.sum(ex, axis=1, keepdims=True)


_head_call = pl.pallas_call(
    _head_body,
    grid=(GRID,),
    in_specs=[
        pl.BlockSpec((ROWS, HD), lambda i: (i, 0)),
        pl.BlockSpec((ROWS, HD), lambda i: (i, 0)),
        pl.BlockSpec((ROWS, 1), lambda i: (i, 0)),
        pl.BlockSpec((ROWS, 1), lambda i: (i, 0)),
        pl.BlockSpec((1, D), lambda i: (0, 0)),
        pl.BlockSpec((D, CLS), lambda i: (0, 0)),
        pl.BlockSpec((1, CLS), lambda i: (0, 0)),
    ],
    out_specs=(pl.BlockSpec((ROWS, CLS), lambda i: (i, 0)),
               pl.BlockSpec((ROWS, CLS), lambda i: (i, 0))),
    out_shape=(jax.ShapeDtypeStruct((N, CLS), jnp.float32),
               jax.ShapeDtypeStruct((N, CLS), jnp.float32)),
)


def kernel(x, edge_index, edge_weight, W1, b1, W2, b2, W_head, b_head):
    src = edge_index[0]
    dst = edge_index[1]
    dstD = dst.reshape(NC * NS, D_CH, D_K)
    ewD = edge_weight.reshape(NC * NS, D_CH, D_K)
    srcA = src.reshape(NS, E // NS)
    dstA = dst.reshape(NS, A_CH, A_K)
    ewA = edge_weight.reshape(NS, A_CH, A_K)
    init_h = jnp.full((N,), 0.5, jnp.float32)

    dega, degb = _deg_call(dstD, ewD, init_h)
    dega = dega.reshape(N, 1)
    degb = degb.reshape(N, 1)

    b1r = b1.reshape(1, D)
    b2r = b2.reshape(1, D)
    bhr = b_head.reshape(1, CLS)

    hp1a, hp1b = _mm1_call(x, W1, dega, degb)
    s1a, s1b = _agg_call(srcA, dstA, ewA, hp1a, hp1b)
    hp2a, hp2b = _mm2_call(s1a, s1b, dega, degb, b1r, W2)
    s2a, s2b = _agg_call(srcA, dstA, ewA, hp2a, hp2b)
    logits, soft = _head_call(s2a, s2b, dega, degb, b2r, W_head, bhr)
    return (logits, soft)
